# Initial kernel scaffold; baseline (speedup 1.0000x reference)
import functools

import jax
import jax.numpy as jnp
from jax import lax
from jax.experimental import pallas as pl
from jax.experimental.pallas import tpu as pltpu
from jax.experimental.pallas import tpu_sc as plsc

N = 10000
E = 320000
D_IN = 128
D_H = 128
D_OUT = 64

NC = 2
NS = 16
NW = NC * NS
EW = E // NW
G = 128
NFULL = EW // G
TAIL = EW - NFULL * G
RSUB = N // NS
ZCH = 125


def _sc_segment_sum(W):
    mesh = plsc.VectorSubcoreMesh(
        core_axis_name="c", subcore_axis_name="s",
        num_cores=NC, num_subcores=NS)

    @functools.partial(
        pl.kernel,
        out_type=jax.ShapeDtypeStruct((NC, N, W), jnp.float32),
        mesh=mesh,
        scratch_types=[
            pltpu.VMEM_SHARED((N, W), jnp.float32),
            pltpu.VMEM((ZCH, W), jnp.float32),
            pltpu.VMEM((G,), jnp.int32),
            pltpu.VMEM((G,), jnp.int32),
            pltpu.VMEM((G, W), jnp.float32),
            pltpu.VMEM((TAIL,), jnp.int32),
            pltpu.VMEM((TAIL,), jnp.int32),
            pltpu.VMEM((TAIL, W), jnp.float32),
        ],
    )
    def k(table, src, dst, out, acc, zbuf, sidx, didx, rows,
          tsidx, tdidx, trows):
        c = lax.axis_index("c")
        s = lax.axis_index("s")
        wid = s * NC + c

        @pl.loop(0, ZCH)
        def _zr(r):
            @pl.loop(0, W // 16)
            def _zc(j):
                zbuf[r, pl.ds(j * 16, 16)] = jnp.zeros((16,), jnp.float32)

        @pl.loop(0, RSUB // ZCH)
        def _zcopy(j):
            pltpu.sync_copy(zbuf, acc.at[pl.ds(s * RSUB + j * ZCH, ZCH)])

        plsc.subcore_barrier()

        base_e = wid * EW

        @pl.loop(0, NFULL)
        def _edges(t):
            e0 = base_e + t * G
            pltpu.sync_copy(src.at[pl.ds(e0, G)], sidx)
            pltpu.sync_copy(dst.at[pl.ds(e0, G)], didx)
            pltpu.sync_copy(table.at[sidx], rows)
            pltpu.sync_copy(rows, acc.at[didx], add=True)

        e0 = base_e + NFULL * G
        pltpu.sync_copy(src.at[pl.ds(e0, TAIL)], tsidx)
        pltpu.sync_copy(dst.at[pl.ds(e0, TAIL)], tdidx)
        pltpu.sync_copy(table.at[tsidx], trows)
        pltpu.sync_copy(trows, acc.at[tdidx], add=True)

        plsc.subcore_barrier()
        pltpu.sync_copy(acc.at[pl.ds(s * RSUB, RSUB)],
                        out.at[c, pl.ds(s * RSUB, RSUB)])

    return k


def _flags(n, width):
    col = lax.broadcasted_iota(jnp.int32, (n, width), 1)
    return jnp.where(col == 0, 1.0, 0.0).astype(jnp.float32)


def _dot(a, b):
    return jnp.dot(a, b, precision=lax.Precision.HIGHEST,
                   preferred_element_type=jnp.float32)


def _tc_prep0_body(x_ref, wn_ref, ws_ref, b_ref, t0_ref, s0_ref):
    x = x_ref[...]
    t0_ref[...] = jnp.concatenate([_dot(x, wn_ref[...]), _flags(N, 16)], 1)
    s0_ref[...] = _dot(x, ws_ref[...]) + b_ref[...]


def _tc_mid_body(s0_ref, p0_ref, wn1_ref, h1_ref, t1_ref):
    p = p0_ref[0] + p0_ref[1]
    deg = jnp.maximum(p[:, D_H:D_H + 1], 1.0)
    h1 = jax.nn.relu(s0_ref[...] + p[:, :D_H] / deg)
    h1_ref[...] = h1
    t1_ref[...] = jnp.concatenate([_dot(h1, wn1_ref[...]), _flags(N, 16)], 1)


def _tc_out_body(h1_ref, p1_ref, ws1_ref, b1_ref, o_ref):
    p = p1_ref[0] + p1_ref[1]
    deg = jnp.maximum(p[:, D_OUT:D_OUT + 1], 1.0)
    o_ref[...] = _dot(h1_ref[...], ws1_ref[...]) + b1_ref[...] \
        + p[:, :D_OUT] / deg


def kernel(x, edge_index0, edge_index1, W_self0, W_neigh0, b0,
           W_self1, W_neigh1, b1):
    src0, dst0 = edge_index0[0], edge_index0[1]
    src1, dst1 = edge_index1[0], edge_index1[1]
    b0r = b0.reshape(1, D_H)
    b1r = b1.reshape(1, D_OUT)

    t0, s0 = pl.pallas_call(
        _tc_prep0_body,
        out_shape=[jax.ShapeDtypeStruct((N, D_H + 16), jnp.float32),
                   jax.ShapeDtypeStruct((N, D_H), jnp.float32)],
    )(x, W_neigh0, W_self0, b0r)

    p0 = _sc_segment_sum(D_H + 16)(t0, src0, dst0)

    h1, t1 = pl.pallas_call(
        _tc_mid_body,
        out_shape=[jax.ShapeDtypeStruct((N, D_H), jnp.float32),
                   jax.ShapeDtypeStruct((N, D_OUT + 16), jnp.float32)],
    )(s0, p0, W_neigh1)

    p1 = _sc_segment_sum(D_OUT + 16)(t1, src1, dst1)

    out = pl.pallas_call(
        _tc_out_body,
        out_shape=jax.ShapeDtypeStruct((N, D_OUT), jnp.float32),
    )(h1, p1, W_self1, b1r)

    return out


# speedup vs baseline: 5.8968x; 5.8968x over previous
import dataclasses
import functools

import jax
import jax.numpy as jnp
from jax import lax
from jax.experimental import pallas as pl
from jax.experimental.pallas import tpu as pltpu
from jax.experimental.pallas import tpu_sc as plsc

N = 10000
E = 320000
D_IN = 128
D_H = 128
D_OUT = 64
W = 128

NC = 2
NS = 16
NW = NC * NS
EW = E // NW
G = 128
NFULL = EW // G
TAIL = EW - NFULL * G
SPAN = 624
ZCH = 104
REM = N - NS * SPAN
REM0 = NS * SPAN
VL = 16


def _sc_segment_sum():
    mesh = plsc.VectorSubcoreMesh(
        core_axis_name="c", subcore_axis_name="s",
        num_cores=NC, num_subcores=NS)
    cp = pltpu.CompilerParams()
    if "needs_layout_passes" in pltpu.CompilerParams.__dataclass_fields__:
        cp = dataclasses.replace(cp, needs_layout_passes=False)

    @functools.partial(
        pl.kernel,
        compiler_params=cp,
        out_type=[jax.ShapeDtypeStruct((NC, N, W), jnp.float32),
                  jax.ShapeDtypeStruct((NW * N,), jnp.float32)],
        mesh=mesh,
        scratch_types=[
            pltpu.VMEM_SHARED((N, W), jnp.float32),
            pltpu.VMEM((ZCH, W), jnp.float32),
            pltpu.VMEM((G,), jnp.int32),
            pltpu.VMEM((G,), jnp.int32),
            pltpu.VMEM((G, W), jnp.float32),
            pltpu.VMEM((TAIL,), jnp.int32),
            pltpu.VMEM((TAIL,), jnp.int32),
            pltpu.VMEM((TAIL, W), jnp.float32),
            pltpu.VMEM((N,), jnp.float32),
        ],
    )
    def k(table, src, dst, out, deg_out, acc, zbuf, sidx, didx, rows,
          tsidx, tdidx, trows, mydeg):
        c = lax.axis_index("c")
        s = lax.axis_index("s")
        wid = s * NC + c

        @pl.loop(0, ZCH)
        def _zr(r):
            @pl.loop(0, W // VL)
            def _zc(j):
                zbuf[r, pl.ds(j * VL, VL)] = jnp.zeros((VL,), jnp.float32)

        @pl.loop(0, N // VL)
        def _zd(i):
            mydeg[pl.ds(i * VL, VL)] = jnp.zeros((VL,), jnp.float32)

        @pl.loop(0, SPAN // ZCH)
        def _zcopy(j):
            pltpu.sync_copy(zbuf, acc.at[pl.ds(s * SPAN + j * ZCH, ZCH)])

        @pl.when(s == 0)
        def _ztail():
            pltpu.sync_copy(zbuf.at[pl.ds(0, REM)], acc.at[pl.ds(REM0, REM)])

        plsc.subcore_barrier()

        base_e = wid * EW

        def count_deg(idx_ref, n):
            for j in range(n // VL):
                dv = idx_ref[pl.ds(j * VL, VL)]
                cnt, last = plsc.scan_count(dv)
                plsc.addupdate_scatter(
                    mydeg, [dv], cnt.astype(jnp.float32), mask=last)

        @pl.loop(0, NFULL)
        def _edges(t):
            e0 = base_e + t * G
            pltpu.sync_copy(src.at[pl.ds(e0, G)], sidx)
            pltpu.sync_copy(dst.at[pl.ds(e0, G)], didx)
            pltpu.sync_copy(table.at[sidx], rows)
            pltpu.sync_copy(rows, acc.at[didx], add=True)
            count_deg(didx, G)

        e0 = base_e + NFULL * G
        pltpu.sync_copy(src.at[pl.ds(e0, TAIL)], tsidx)
        pltpu.sync_copy(dst.at[pl.ds(e0, TAIL)], tdidx)
        pltpu.sync_copy(table.at[tsidx], trows)
        pltpu.sync_copy(trows, acc.at[tdidx], add=True)
        count_deg(tdidx, TAIL)

        plsc.subcore_barrier()
        pltpu.sync_copy(acc.at[pl.ds(s * SPAN, SPAN)],
                        out.at[c, pl.ds(s * SPAN, SPAN)])

        @pl.when(s == 0)
        def _otail():
            pltpu.sync_copy(acc.at[pl.ds(REM0, REM)],
                            out.at[c, pl.ds(REM0, REM)])

        pltpu.sync_copy(mydeg, deg_out.at[pl.ds(wid * N, N)])

    return k


def _dot(a, b):
    return jnp.dot(a, b, precision=lax.Precision.HIGHEST,
                   preferred_element_type=jnp.float32)


BN = 2000


def _tc_call(body, d_in, d_out):
    return pl.pallas_call(
        body,
        grid=(N // BN,),
        in_specs=[
            pl.BlockSpec((BN, d_in), lambda i: (i, 0)),
            pl.BlockSpec((NC, BN, W), lambda i: (0, i, 0)),
            pl.BlockSpec((BN, NW), lambda i: (i, 0)),
            pl.BlockSpec((d_in, d_out), lambda i: (0, 0)),
            pl.BlockSpec((d_in, d_out), lambda i: (0, 0)),
            pl.BlockSpec((1, d_out), lambda i: (0, 0)),
        ],
        out_specs=pl.BlockSpec((BN, d_out), lambda i: (i, 0)),
        out_shape=jax.ShapeDtypeStruct((N, d_out), jnp.float32),
    )


def _mean(p_ref, dp_ref, d):
    agg = p_ref[0] + p_ref[1]
    deg = jnp.maximum(jnp.sum(dp_ref[...], axis=1, keepdims=True), 1.0)
    return agg[:, :d] / deg


def _tc_mid_body(x_ref, p0_ref, dp0_ref, ws_ref, wn_ref, b_ref, h1_ref):
    mean = _mean(p0_ref, dp0_ref, D_IN)
    h1_ref[...] = jax.nn.relu(
        _dot(x_ref[...], ws_ref[...]) + _dot(mean, wn_ref[...]) + b_ref[...])


def _tc_out_body(h1_ref, p1_ref, dp1_ref, ws_ref, wn_ref, b_ref, o_ref):
    mean = _mean(p1_ref, dp1_ref, D_H)
    o_ref[...] = (_dot(h1_ref[...], ws_ref[...]) + _dot(mean, wn_ref[...])
                  + b_ref[...])


def kernel(x, edge_index0, edge_index1, W_self0, W_neigh0, b0,
           W_self1, W_neigh1, b1):
    src0, dst0 = edge_index0[0], edge_index0[1]
    src1, dst1 = edge_index1[0], edge_index1[1]
    b0r = b0.reshape(1, D_H)
    b1r = b1.reshape(1, D_OUT)
    sc = _sc_segment_sum()

    p0, degf0 = sc(x, src0, dst0)
    dp0 = degf0.reshape(NW, N).T

    h1 = _tc_call(_tc_mid_body, D_IN, D_H)(x, p0, dp0, W_self0, W_neigh0, b0r)

    p1, degf1 = sc(h1, src1, dst1)
    dp1 = degf1.reshape(NW, N).T

    out = _tc_call(_tc_out_body, D_H, D_OUT)(h1, p1, dp1, W_self1, W_neigh1,
                                             b1r)

    return out


# speedup vs baseline: 7.7747x; 1.3185x over previous
import dataclasses
import functools

import jax
import jax.numpy as jnp
from jax import lax
from jax.experimental import pallas as pl
from jax.experimental.pallas import tpu as pltpu
from jax.experimental.pallas import tpu_sc as plsc

N = 10000
E = 320000
D_IN = 128
D_H = 128
D_OUT = 64
W = 128

NC = 2
NS = 16
NW = NC * NS
EW = E // NW
G = 64
NFULL = EW // G
TAIL = EW - NFULL * G
SPAN = 624
ZCH = 104
REM = N - NS * SPAN
REM0 = NS * SPAN
VL = 16
NB = 4
AHEAD = 2
NMID = ((NFULL - AHEAD - NB) // NB) * NB


def _sc_segment_sum():
    mesh = plsc.VectorSubcoreMesh(
        core_axis_name="c", subcore_axis_name="s",
        num_cores=NC, num_subcores=NS)
    cp = pltpu.CompilerParams()
    if "needs_layout_passes" in pltpu.CompilerParams.__dataclass_fields__:
        cp = dataclasses.replace(cp, needs_layout_passes=False)

    @functools.partial(
        pl.kernel,
        compiler_params=cp,
        out_type=[jax.ShapeDtypeStruct((NC, N, W), jnp.float32),
                  jax.ShapeDtypeStruct((NW * N,), jnp.float32)],
        mesh=mesh,
        scratch_types=[
            pltpu.VMEM_SHARED((N, W), jnp.float32),
            [pltpu.VMEM((G,), jnp.int32)] * NB,
            [pltpu.VMEM((G,), jnp.int32)] * NB,
            [pltpu.VMEM((G, W), jnp.float32)] * NB,
            pltpu.VMEM((TAIL,), jnp.int32),
            pltpu.VMEM((TAIL,), jnp.int32),
            pltpu.VMEM((TAIL, W), jnp.float32),
            pltpu.VMEM((N,), jnp.float32),
            [pltpu.SemaphoreType.DMA] * NB,
            [pltpu.SemaphoreType.DMA] * NB,
        ],
    )
    def k(table, src, dst, zhbm, out, deg_out, acc, sidx, didx, rows,
          tsidx, tdidx, trows, mydeg, gsem, ssem):
        c = lax.axis_index("c")
        s = lax.axis_index("s")
        wid = s * NC + c

        @pl.loop(0, N // VL)
        def _zd(i):
            mydeg[pl.ds(i * VL, VL)] = jnp.zeros((VL,), jnp.float32)

        @pl.loop(0, SPAN // ZCH)
        def _zcopy(j):
            pltpu.sync_copy(zhbm, acc.at[pl.ds(s * SPAN + j * ZCH, ZCH)])

        @pl.when(s == 0)
        def _ztail():
            pltpu.sync_copy(zhbm.at[pl.ds(0, REM)], acc.at[pl.ds(REM0, REM)])

        plsc.subcore_barrier()

        base_e = wid * EW

        def count_deg(idx_ref, n):
            for j in range(n // VL):
                dv = idx_ref[pl.ds(j * VL, VL)]
                cnt, last = plsc.scan_count(dv)
                plsc.addupdate_scatter(
                    mydeg, [dv], cnt.astype(jnp.float32), mask=last)

        def load_and_gather(g, b):
            e0 = base_e + g * G
            pltpu.sync_copy(src.at[pl.ds(e0, G)], sidx[b])
            pltpu.sync_copy(dst.at[pl.ds(e0, G)], didx[b])
            pltpu.async_copy(table.at[sidx[b]], rows[b], gsem[b])

        def step(g, b, issue, wait_prev):
            b2 = (b + AHEAD) % NB
            if issue:
                if wait_prev:
                    pltpu.make_async_copy(
                        rows[b2], acc.at[didx[b2]], ssem[b2]).wait()
                load_and_gather(g + AHEAD, b2)
            pltpu.make_async_copy(table.at[sidx[b]], rows[b], gsem[b]).wait()
            pltpu.async_copy(rows[b], acc.at[didx[b]], ssem[b], add=True)
            count_deg(didx[b], G)

        load_and_gather(0, 0)
        load_and_gather(1, 1)
        step(0, 0, True, False)
        step(1, 1, True, False)

        @pl.loop(AHEAD, AHEAD + NMID, step=NB)
        def _edges(t):
            for db in range(NB):
                step(t + db, (AHEAD + db) % NB, True, True)

        for g in range(AHEAD + NMID, NFULL):
            step(g, g % NB, g + AHEAD < NFULL, True)
        for q in range(NFULL - NB, NFULL):
            pltpu.make_async_copy(
                rows[q % NB], acc.at[didx[q % NB]], ssem[q % NB]).wait()

        e0 = base_e + NFULL * G
        pltpu.sync_copy(src.at[pl.ds(e0, TAIL)], tsidx)
        pltpu.sync_copy(dst.at[pl.ds(e0, TAIL)], tdidx)
        pltpu.sync_copy(table.at[tsidx], trows)
        pltpu.sync_copy(trows, acc.at[tdidx], add=True)
        count_deg(tdidx, TAIL)

        plsc.subcore_barrier()
        pltpu.sync_copy(acc.at[pl.ds(s * SPAN, SPAN)],
                        out.at[c, pl.ds(s * SPAN, SPAN)])

        @pl.when(s == 0)
        def _otail():
            pltpu.sync_copy(acc.at[pl.ds(REM0, REM)],
                            out.at[c, pl.ds(REM0, REM)])

        pltpu.sync_copy(mydeg, deg_out.at[pl.ds(wid * N, N)])

    return k


def _dot(a, b):
    return jnp.dot(a, b, precision=lax.Precision.HIGHEST,
                   preferred_element_type=jnp.float32)


BN = 2000


def _tc_call(body, d_in, d_out):
    return pl.pallas_call(
        body,
        grid=(N // BN,),
        in_specs=[
            pl.BlockSpec((BN, d_in), lambda i: (i, 0)),
            pl.BlockSpec((NC, BN, W), lambda i: (0, i, 0)),
            pl.BlockSpec((BN, NW), lambda i: (i, 0)),
            pl.BlockSpec((d_in, d_out), lambda i: (0, 0)),
            pl.BlockSpec((d_in, d_out), lambda i: (0, 0)),
            pl.BlockSpec((1, d_out), lambda i: (0, 0)),
        ],
        out_specs=pl.BlockSpec((BN, d_out), lambda i: (i, 0)),
        out_shape=jax.ShapeDtypeStruct((N, d_out), jnp.float32),
    )


def _mean(p_ref, dp_ref, d):
    agg = p_ref[0] + p_ref[1]
    deg = jnp.maximum(jnp.sum(dp_ref[...], axis=1, keepdims=True), 1.0)
    return agg[:, :d] / deg


def _tc_mid_body(x_ref, p0_ref, dp0_ref, ws_ref, wn_ref, b_ref, h1_ref):
    mean = _mean(p0_ref, dp0_ref, D_IN)
    h1_ref[...] = jax.nn.relu(
        _dot(x_ref[...], ws_ref[...]) + _dot(mean, wn_ref[...]) + b_ref[...])


def _tc_out_body(h1_ref, p1_ref, dp1_ref, ws_ref, wn_ref, b_ref, o_ref):
    mean = _mean(p1_ref, dp1_ref, D_H)
    o_ref[...] = (_dot(h1_ref[...], ws_ref[...]) + _dot(mean, wn_ref[...])
                  + b_ref[...])


def kernel(x, edge_index0, edge_index1, W_self0, W_neigh0, b0,
           W_self1, W_neigh1, b1):
    src0, dst0 = edge_index0[0], edge_index0[1]
    src1, dst1 = edge_index1[0], edge_index1[1]
    b0r = b0.reshape(1, D_H)
    b1r = b1.reshape(1, D_OUT)
    sc = _sc_segment_sum()

    zs = jnp.zeros((ZCH, W), jnp.float32)
    p0, degf0 = sc(x, src0, dst0, zs)
    dp0 = degf0.reshape(NW, N).T

    h1 = _tc_call(_tc_mid_body, D_IN, D_H)(x, p0, dp0, W_self0, W_neigh0, b0r)

    p1, degf1 = sc(h1, src1, dst1, zs)
    dp1 = degf1.reshape(NW, N).T

    out = _tc_call(_tc_out_body, D_H, D_OUT)(h1, p1, dp1, W_self1, W_neigh1,
                                             b1r)

    return out


# speedup vs baseline: 10.4106x; 1.3390x over previous
import dataclasses
import functools

import jax
import jax.numpy as jnp
from jax import lax
from jax.experimental import pallas as pl
from jax.experimental.pallas import tpu as pltpu
from jax.experimental.pallas import tpu_sc as plsc

N = 10000
E = 320000
D_IN = 128
D_H = 128
D_OUT = 64
W = 128

NC = 2
NS = 16
NW = NC * NS
EW = E // NW
G = 64
NFULL = EW // G
TAIL = EW - NFULL * G
SPAN = 624
ZCH = 104
REM = N - NS * SPAN
REM0 = NS * SPAN
VL = 16
NB = 4
AHEAD = 2
CH = 12
CHW = CH * G
NCHUNK = EW // CHW
UNROLL = 2 * CH
NMID = 6 * UNROLL


def _sc_segment_sum():
    mesh = plsc.VectorSubcoreMesh(
        core_axis_name="c", subcore_axis_name="s",
        num_cores=NC, num_subcores=NS)
    cp = pltpu.CompilerParams()
    if "needs_layout_passes" in pltpu.CompilerParams.__dataclass_fields__:
        cp = dataclasses.replace(cp, needs_layout_passes=False)

    @functools.partial(
        pl.kernel,
        compiler_params=cp,
        out_type=[jax.ShapeDtypeStruct((NC, N, W), jnp.float32),
                  jax.ShapeDtypeStruct((NW * N,), jnp.float32)],
        mesh=mesh,
        scratch_types=[
            pltpu.VMEM_SHARED((N, W), jnp.float32),
            [pltpu.VMEM((CHW,), jnp.int32)] * 2,
            [pltpu.VMEM((CHW,), jnp.int32)] * 2,
            [pltpu.VMEM((G, W), jnp.float32)] * NB,
            pltpu.VMEM((TAIL,), jnp.int32),
            pltpu.VMEM((TAIL,), jnp.int32),
            pltpu.VMEM((TAIL, W), jnp.float32),
            pltpu.VMEM((N,), jnp.float32),
            [pltpu.SemaphoreType.DMA] * NB,
            [pltpu.SemaphoreType.DMA] * NB,
        ],
    )
    def k(table, src, dst, zhbm, out, deg_out, acc, scnk, dcnk, rows,
          tsidx, tdidx, trows, mydeg, gsem, ssem):
        c = lax.axis_index("c")
        s = lax.axis_index("s")
        wid = s * NC + c

        @pl.loop(0, N // VL)
        def _zd(i):
            mydeg[pl.ds(i * VL, VL)] = jnp.zeros((VL,), jnp.float32)

        @pl.loop(0, SPAN // ZCH)
        def _zcopy(j):
            pltpu.sync_copy(zhbm, acc.at[pl.ds(s * SPAN + j * ZCH, ZCH)])

        @pl.when(s == 0)
        def _ztail():
            pltpu.sync_copy(zhbm.at[pl.ds(0, REM)], acc.at[pl.ds(REM0, REM)])

        plsc.subcore_barrier()

        base_e = wid * EW

        def count_deg_tail(idx_ref, n):
            for j in range(n // VL):
                dv = idx_ref[pl.ds(j * VL, VL)]
                cnt, last = plsc.scan_count(dv)
                plsc.addupdate_scatter(
                    mydeg, [dv], cnt.astype(jnp.float32), mask=last)

        def count_deg(cb, co):
            for j in range(G // VL):
                dv = dcnk[cb][pl.ds(co * G + j * VL, VL)]
                cnt, last = plsc.scan_count(dv)
                plsc.addupdate_scatter(
                    mydeg, [dv], cnt.astype(jnp.float32), mask=last)

        def load_chunk(buf, e0):
            pltpu.sync_copy(src.at[pl.ds(e0, CHW)], scnk[buf])
            pltpu.sync_copy(dst.at[pl.ds(e0, CHW)], dcnk[buf])

        def issue_gather(slot, qb, qo):
            pltpu.async_copy(table.at[scnk[qb].at[pl.ds(qo * G, G)]],
                             rows[slot], gsem[slot])

        def wait_gather(slot):
            pltpu.make_async_copy(table.at[scnk[0].at[pl.ds(0, G)]],
                                  rows[slot], gsem[slot]).wait()

        def issue_scatter(slot, cb, co):
            pltpu.async_copy(rows[slot],
                             acc.at[dcnk[cb].at[pl.ds(co * G, G)]],
                             ssem[slot], add=True)

        def wait_scatter(slot):
            pltpu.make_async_copy(rows[slot],
                                  acc.at[dcnk[0].at[pl.ds(0, G)]],
                                  ssem[slot]).wait()

        def step(slot, cb, co, qb, qo, issue, wait_prev):
            if issue:
                slot2 = (slot + AHEAD) % NB
                if wait_prev:
                    wait_scatter(slot2)
                issue_gather(slot2, qb, qo)
            wait_gather(slot)
            issue_scatter(slot, cb, co)
            count_deg(cb, co)

        load_chunk(0, base_e)
        issue_gather(0, 0, 0)
        issue_gather(1, 0, 1)
        step(0, 0, 0, 0, 2, True, False)
        step(1, 0, 1, 0, 3, True, False)

        @pl.loop(AHEAD, AHEAD + NMID, step=UNROLL)
        def _edges(t):
            chunk_e0 = base_e + ((t - AHEAD) // CH + 1) * CHW
            for db in range(UNROLL):
                if db == 0:
                    load_chunk(1, chunk_e0)
                if db == CH:
                    load_chunk(0, chunk_e0 + CHW)
                slot = (AHEAD + db) % NB
                cb, co = ((AHEAD + db) // CH) % 2, (AHEAD + db) % CH
                qb, qo = ((2 * AHEAD + db) // CH) % 2, (2 * AHEAD + db) % CH
                step(slot, cb, co, qb, qo, True, True)

        for g in range(AHEAD + NMID, NFULL):
            q = g + AHEAD
            step(g % NB, (g // CH) % 2, g % CH, (q // CH) % 2, q % CH,
                 q < NFULL, True)
        for q in range(NFULL - NB, NFULL):
            wait_scatter(q % NB)

        e0 = base_e + NFULL * G
        pltpu.sync_copy(src.at[pl.ds(e0, TAIL)], tsidx)
        pltpu.sync_copy(dst.at[pl.ds(e0, TAIL)], tdidx)
        pltpu.sync_copy(table.at[tsidx], trows)
        pltpu.sync_copy(trows, acc.at[tdidx], add=True)
        count_deg_tail(tdidx, TAIL)

        plsc.subcore_barrier()
        pltpu.sync_copy(acc.at[pl.ds(s * SPAN, SPAN)],
                        out.at[c, pl.ds(s * SPAN, SPAN)])

        @pl.when(s == 0)
        def _otail():
            pltpu.sync_copy(acc.at[pl.ds(REM0, REM)],
                            out.at[c, pl.ds(REM0, REM)])

        pltpu.sync_copy(mydeg, deg_out.at[pl.ds(wid * N, N)])

    return k


def _dot(a, b):
    return jnp.dot(a, b, precision=lax.Precision.HIGHEST,
                   preferred_element_type=jnp.float32)


BN = 2000


def _tc_call(body, d_in, d_out):
    return pl.pallas_call(
        body,
        grid=(N // BN,),
        in_specs=[
            pl.BlockSpec((BN, d_in), lambda i: (i, 0)),
            pl.BlockSpec((NC, BN, W), lambda i: (0, i, 0)),
            pl.BlockSpec((BN, NW), lambda i: (i, 0)),
            pl.BlockSpec((d_in, d_out), lambda i: (0, 0)),
            pl.BlockSpec((d_in, d_out), lambda i: (0, 0)),
            pl.BlockSpec((1, d_out), lambda i: (0, 0)),
        ],
        out_specs=pl.BlockSpec((BN, d_out), lambda i: (i, 0)),
        out_shape=jax.ShapeDtypeStruct((N, d_out), jnp.float32),
    )


def _mean(p_ref, dp_ref, d):
    agg = p_ref[0] + p_ref[1]
    deg = jnp.maximum(jnp.sum(dp_ref[...], axis=1, keepdims=True), 1.0)
    return agg[:, :d] / deg


def _tc_mid_body(x_ref, p0_ref, dp0_ref, ws_ref, wn_ref, b_ref, h1_ref):
    mean = _mean(p0_ref, dp0_ref, D_IN)
    h1_ref[...] = jax.nn.relu(
        _dot(x_ref[...], ws_ref[...]) + _dot(mean, wn_ref[...]) + b_ref[...])


def _tc_out_body(h1_ref, p1_ref, dp1_ref, ws_ref, wn_ref, b_ref, o_ref):
    mean = _mean(p1_ref, dp1_ref, D_H)
    o_ref[...] = (_dot(h1_ref[...], ws_ref[...]) + _dot(mean, wn_ref[...])
                  + b_ref[...])


def kernel(x, edge_index0, edge_index1, W_self0, W_neigh0, b0,
           W_self1, W_neigh1, b1):
    src0, dst0 = edge_index0[0], edge_index0[1]
    src1, dst1 = edge_index1[0], edge_index1[1]
    b0r = b0.reshape(1, D_H)
    b1r = b1.reshape(1, D_OUT)
    sc = _sc_segment_sum()

    zs = jnp.zeros((ZCH, W), jnp.float32)
    p0, degf0 = sc(x, src0, dst0, zs)
    dp0 = degf0.reshape(NW, N).T

    h1 = _tc_call(_tc_mid_body, D_IN, D_H)(x, p0, dp0, W_self0, W_neigh0, b0r)

    p1, degf1 = sc(h1, src1, dst1, zs)
    dp1 = degf1.reshape(NW, N).T

    out = _tc_call(_tc_out_body, D_H, D_OUT)(h1, p1, dp1, W_self1, W_neigh1,
                                             b1r)

    return out


# speedup vs baseline: 10.6810x; 1.0260x over previous
import dataclasses
import functools

import jax
import jax.numpy as jnp
from jax import lax
from jax.experimental import pallas as pl
from jax.experimental.pallas import tpu as pltpu
from jax.experimental.pallas import tpu_sc as plsc

N = 10000
E = 320000
D_IN = 128
D_H = 128
D_OUT = 64
W = 128

NC = 2
NS = 16
NW = NC * NS
EW = E // NW
G = 64
NFULL = EW // G
TAIL = EW - NFULL * G
SPAN = 624
ZCH = 104
REM = N - NS * SPAN
REM0 = NS * SPAN
VL = 16
NB = 4
AHEAD = 2
CH = 12
CHW = CH * G
NCHUNK = EW // CHW
UNROLL = 2 * CH
NMID = 6 * UNROLL


def _sc_segment_sum():
    mesh = plsc.VectorSubcoreMesh(
        core_axis_name="c", subcore_axis_name="s",
        num_cores=NC, num_subcores=NS)
    cp = pltpu.CompilerParams()
    if "needs_layout_passes" in pltpu.CompilerParams.__dataclass_fields__:
        cp = dataclasses.replace(cp, needs_layout_passes=False)

    @functools.partial(
        pl.kernel,
        compiler_params=cp,
        out_type=[jax.ShapeDtypeStruct((NC, N, W), jnp.float32),
                  jax.ShapeDtypeStruct((NW * N,), jnp.float32)],
        mesh=mesh,
        scratch_types=[
            pltpu.VMEM_SHARED((N, W), jnp.float32),
            [pltpu.VMEM((CHW,), jnp.int32)] * 2,
            [pltpu.VMEM((CHW,), jnp.int32)] * 2,
            [pltpu.VMEM((G, W), jnp.float32)] * NB,
            pltpu.VMEM((TAIL,), jnp.int32),
            pltpu.VMEM((TAIL,), jnp.int32),
            pltpu.VMEM((TAIL, W), jnp.float32),
            pltpu.VMEM((N,), jnp.float32),
            [pltpu.SemaphoreType.DMA] * NB,
            [pltpu.SemaphoreType.DMA] * NB,
            pltpu.SemaphoreType.DMA,
        ],
    )
    def k(table, src, dst, zhbm, out, deg_out, acc, scnk, dcnk, rows,
          tsidx, tdidx, trows, mydeg, gsem, ssem, zsem):
        c = lax.axis_index("c")
        s = lax.axis_index("s")
        wid = s * NC + c
        base_e = wid * EW

        def count_deg_tail(idx_ref, n):
            for j in range(n // VL):
                dv = idx_ref[pl.ds(j * VL, VL)]
                cnt, last = plsc.scan_count(dv)
                plsc.addupdate_scatter(
                    mydeg, [dv], cnt.astype(jnp.float32), mask=last)

        def count_deg(cb, co):
            for j in range(G // VL):
                dv = dcnk[cb][pl.ds(co * G + j * VL, VL)]
                cnt, last = plsc.scan_count(dv)
                plsc.addupdate_scatter(
                    mydeg, [dv], cnt.astype(jnp.float32), mask=last)

        def load_chunk(buf, e0):
            pltpu.sync_copy(src.at[pl.ds(e0, CHW)], scnk[buf])
            pltpu.sync_copy(dst.at[pl.ds(e0, CHW)], dcnk[buf])

        def issue_gather(slot, qb, qo):
            pltpu.async_copy(table.at[scnk[qb].at[pl.ds(qo * G, G)]],
                             rows[slot], gsem[slot])

        def wait_gather(slot):
            pltpu.make_async_copy(table.at[scnk[0].at[pl.ds(0, G)]],
                                  rows[slot], gsem[slot]).wait()

        def issue_scatter(slot, cb, co):
            pltpu.async_copy(rows[slot],
                             acc.at[dcnk[cb].at[pl.ds(co * G, G)]],
                             ssem[slot], add=True)

        def wait_scatter(slot):
            pltpu.make_async_copy(rows[slot],
                                  acc.at[dcnk[0].at[pl.ds(0, G)]],
                                  ssem[slot]).wait()

        def step(slot, cb, co, qb, qo, issue, wait_prev):
            if issue:
                slot2 = (slot + AHEAD) % NB
                if wait_prev:
                    wait_scatter(slot2)
                issue_gather(slot2, qb, qo)
            wait_gather(slot)
            issue_scatter(slot, cb, co)
            count_deg(cb, co)

        load_chunk(0, base_e)
        issue_gather(0, 0, 0)
        issue_gather(1, 0, 1)

        zplan = [(s * SPAN + j * ZCH, ZCH) for j in range(SPAN // ZCH)]
        for r0, nr in zplan:
            pltpu.async_copy(zhbm, acc.at[pl.ds(r0, nr)], zsem)

        @pl.when(s == 0)
        def _ztail():
            pltpu.async_copy(zhbm.at[pl.ds(0, REM)],
                             acc.at[pl.ds(REM0, REM)], zsem)

        @pl.loop(0, N // VL)
        def _zd(i):
            mydeg[pl.ds(i * VL, VL)] = jnp.zeros((VL,), jnp.float32)

        for r0, nr in zplan:
            pltpu.make_async_copy(zhbm, acc.at[pl.ds(r0, nr)], zsem).wait()

        @pl.when(s == 0)
        def _ztailw():
            pltpu.make_async_copy(zhbm.at[pl.ds(0, REM)],
                                  acc.at[pl.ds(REM0, REM)], zsem).wait()

        plsc.subcore_barrier()

        step(0, 0, 0, 0, 2, True, False)
        step(1, 0, 1, 0, 3, True, False)

        @pl.loop(AHEAD, AHEAD + NMID, step=UNROLL)
        def _edges(t):
            chunk_e0 = base_e + ((t - AHEAD) // CH + 1) * CHW
            for db in range(UNROLL):
                if db == 0:
                    load_chunk(1, chunk_e0)
                if db == CH:
                    load_chunk(0, chunk_e0 + CHW)
                slot = (AHEAD + db) % NB
                cb, co = ((AHEAD + db) // CH) % 2, (AHEAD + db) % CH
                qb, qo = ((2 * AHEAD + db) // CH) % 2, (2 * AHEAD + db) % CH
                step(slot, cb, co, qb, qo, True, True)

        for g in range(AHEAD + NMID, NFULL):
            q = g + AHEAD
            step(g % NB, (g // CH) % 2, g % CH, (q // CH) % 2, q % CH,
                 q < NFULL, True)
        for q in range(NFULL - NB, NFULL):
            wait_scatter(q % NB)

        e0 = base_e + NFULL * G
        pltpu.sync_copy(src.at[pl.ds(e0, TAIL)], tsidx)
        pltpu.sync_copy(dst.at[pl.ds(e0, TAIL)], tdidx)
        pltpu.sync_copy(table.at[tsidx], trows)
        pltpu.sync_copy(trows, acc.at[tdidx], add=True)
        count_deg_tail(tdidx, TAIL)

        plsc.subcore_barrier()
        pltpu.sync_copy(acc.at[pl.ds(s * SPAN, SPAN)],
                        out.at[c, pl.ds(s * SPAN, SPAN)])

        @pl.when(s == 0)
        def _otail():
            pltpu.sync_copy(acc.at[pl.ds(REM0, REM)],
                            out.at[c, pl.ds(REM0, REM)])

        pltpu.sync_copy(mydeg, deg_out.at[pl.ds(wid * N, N)])

    return k


def _dot(a, b):
    return jnp.dot(a, b, precision=lax.Precision.HIGHEST,
                   preferred_element_type=jnp.float32)


BN = 2000


def _tc_call(body, d_in, d_out):
    return pl.pallas_call(
        body,
        grid=(N // BN,),
        in_specs=[
            pl.BlockSpec((BN, d_in), lambda i: (i, 0)),
            pl.BlockSpec((NC, BN, W), lambda i: (0, i, 0)),
            pl.BlockSpec((BN, NW), lambda i: (i, 0)),
            pl.BlockSpec((d_in, d_out), lambda i: (0, 0)),
            pl.BlockSpec((d_in, d_out), lambda i: (0, 0)),
            pl.BlockSpec((1, d_out), lambda i: (0, 0)),
        ],
        out_specs=pl.BlockSpec((BN, d_out), lambda i: (i, 0)),
        out_shape=jax.ShapeDtypeStruct((N, d_out), jnp.float32),
    )


def _mean(p_ref, dp_ref, d):
    agg = p_ref[0] + p_ref[1]
    deg = jnp.maximum(jnp.sum(dp_ref[...], axis=1, keepdims=True), 1.0)
    return agg[:, :d] / deg


def _tc_mid_body(x_ref, p0_ref, dp0_ref, ws_ref, wn_ref, b_ref, h1_ref):
    mean = _mean(p0_ref, dp0_ref, D_IN)
    h1_ref[...] = jax.nn.relu(
        _dot(x_ref[...], ws_ref[...]) + _dot(mean, wn_ref[...]) + b_ref[...])


def _tc_out_body(h1_ref, p1_ref, dp1_ref, ws_ref, wn_ref, b_ref, o_ref):
    mean = _mean(p1_ref, dp1_ref, D_H)
    o_ref[...] = (_dot(h1_ref[...], ws_ref[...]) + _dot(mean, wn_ref[...])
                  + b_ref[...])


def kernel(x, edge_index0, edge_index1, W_self0, W_neigh0, b0,
           W_self1, W_neigh1, b1):
    src0, dst0 = edge_index0[0], edge_index0[1]
    src1, dst1 = edge_index1[0], edge_index1[1]
    b0r = b0.reshape(1, D_H)
    b1r = b1.reshape(1, D_OUT)
    sc = _sc_segment_sum()

    zs = jnp.zeros((ZCH, W), jnp.float32)
    p0, degf0 = sc(x, src0, dst0, zs)
    dp0 = degf0.reshape(NW, N).T

    h1 = _tc_call(_tc_mid_body, D_IN, D_H)(x, p0, dp0, W_self0, W_neigh0, b0r)

    p1, degf1 = sc(h1, src1, dst1, zs)
    dp1 = degf1.reshape(NW, N).T

    out = _tc_call(_tc_out_body, D_H, D_OUT)(h1, p1, dp1, W_self1, W_neigh1,
                                             b1r)

    return out


# speedup vs baseline: 11.2132x; 1.0498x over previous
import dataclasses
import functools

import jax
import jax.numpy as jnp
from jax import lax
from jax.experimental import pallas as pl
from jax.experimental.pallas import tpu as pltpu
from jax.experimental.pallas import tpu_sc as plsc

N = 10000
E = 320000
D_IN = 128
D_H = 128
D_OUT = 64
W = 128

NC = 2
NS = 16
NW = NC * NS
EW = E // NW
G = 64
NFULL = EW // G
TAIL = EW - NFULL * G
SPAN = 624
ZCH = 104
REM = N - NS * SPAN
REM0 = NS * SPAN
VL = 16
NB = 4
AHEAD = 3
CH = 12
CHW = CH * G
NCHUNK = EW // CHW
UNROLL = 2 * CH
T0 = 15
TE = T0 + 5 * UNROLL


def _sc_segment_sum():
    mesh = plsc.VectorSubcoreMesh(
        core_axis_name="c", subcore_axis_name="s",
        num_cores=NC, num_subcores=NS)
    cp = pltpu.CompilerParams()
    if "needs_layout_passes" in pltpu.CompilerParams.__dataclass_fields__:
        cp = dataclasses.replace(cp, needs_layout_passes=False)

    @functools.partial(
        pl.kernel,
        compiler_params=cp,
        out_type=[jax.ShapeDtypeStruct((NC, N, W), jnp.float32),
                  jax.ShapeDtypeStruct((NW * N,), jnp.float32)],
        mesh=mesh,
        scratch_types=[
            pltpu.VMEM_SHARED((N, W), jnp.float32),
            [pltpu.VMEM((CHW,), jnp.int32)] * 2,
            [pltpu.VMEM((CHW,), jnp.int32)] * 2,
            [pltpu.VMEM((G, W), jnp.float32)] * NB,
            pltpu.VMEM((TAIL,), jnp.int32),
            pltpu.VMEM((TAIL,), jnp.int32),
            pltpu.VMEM((TAIL, W), jnp.float32),
            pltpu.VMEM((N,), jnp.float32),
            [pltpu.SemaphoreType.DMA] * NB,
            [pltpu.SemaphoreType.DMA] * NB,
            pltpu.SemaphoreType.DMA,
        ],
    )
    def k(table, src, dst, zhbm, out, deg_out, acc, scnk, dcnk, rows,
          tsidx, tdidx, trows, mydeg, gsem, ssem, zsem):
        c = lax.axis_index("c")
        s = lax.axis_index("s")
        wid = s * NC + c
        base_e = wid * EW

        def count_deg_tail(idx_ref, n):
            for j in range(n // VL):
                dv = idx_ref[pl.ds(j * VL, VL)]
                cnt, last = plsc.scan_count(dv)
                plsc.addupdate_scatter(
                    mydeg, [dv], cnt.astype(jnp.float32), mask=last)

        def count_deg(cb, co):
            for j in range(G // VL):
                dv = dcnk[cb][pl.ds(co * G + j * VL, VL)]
                cnt, last = plsc.scan_count(dv)
                plsc.addupdate_scatter(
                    mydeg, [dv], cnt.astype(jnp.float32), mask=last)

        def load_chunk(buf, e0):
            pltpu.sync_copy(src.at[pl.ds(e0, CHW)], scnk[buf])
            pltpu.sync_copy(dst.at[pl.ds(e0, CHW)], dcnk[buf])

        def issue_gather(slot, qb, qo):
            pltpu.async_copy(table.at[scnk[qb].at[pl.ds(qo * G, G)]],
                             rows[slot], gsem[slot])

        def wait_gather(slot):
            pltpu.make_async_copy(table.at[scnk[0].at[pl.ds(0, G)]],
                                  rows[slot], gsem[slot]).wait()

        def issue_scatter(slot, cb, co):
            pltpu.async_copy(rows[slot],
                             acc.at[dcnk[cb].at[pl.ds(co * G, G)]],
                             ssem[slot], add=True)

        def wait_scatter(slot):
            pltpu.make_async_copy(rows[slot],
                                  acc.at[dcnk[0].at[pl.ds(0, G)]],
                                  ssem[slot]).wait()

        def step(slot, cb, co, qb, qo, issue, wait_prev):
            if issue:
                slot2 = (slot + AHEAD) % NB
                if wait_prev:
                    wait_scatter(slot2)
                issue_gather(slot2, qb, qo)
            wait_gather(slot)
            issue_scatter(slot, cb, co)
            count_deg(cb, co)

        def emit(g):
            if g % CH == 2 and g // CH + 1 < NCHUNK:
                load_chunk((g // CH + 1) % 2,
                           base_e + (g // CH + 1) * CHW)
            q = g + AHEAD
            step(g % NB, (g // CH) % 2, g % CH, (q // CH) % 2, q % CH,
                 q < NFULL, q >= NB)

        load_chunk(0, base_e)
        for q in range(AHEAD):
            issue_gather(q % NB, 0, q)

        zplan = [(s * SPAN + j * ZCH, ZCH) for j in range(SPAN // ZCH)]
        for r0, nr in zplan:
            pltpu.async_copy(zhbm, acc.at[pl.ds(r0, nr)], zsem)

        @pl.when(s == 0)
        def _ztail():
            pltpu.async_copy(zhbm.at[pl.ds(0, REM)],
                             acc.at[pl.ds(REM0, REM)], zsem)

        @pl.loop(0, N // VL)
        def _zd(i):
            mydeg[pl.ds(i * VL, VL)] = jnp.zeros((VL,), jnp.float32)

        for r0, nr in zplan:
            pltpu.make_async_copy(zhbm, acc.at[pl.ds(r0, nr)], zsem).wait()

        @pl.when(s == 0)
        def _ztailw():
            pltpu.make_async_copy(zhbm.at[pl.ds(0, REM)],
                                  acc.at[pl.ds(REM0, REM)], zsem).wait()

        plsc.subcore_barrier()

        for g in range(T0):
            emit(g)

        @pl.loop(T0, TE, step=UNROLL)
        def _edges(t):
            for db in range(UNROLL):
                gg = T0 + db
                if gg % CH == 2:
                    load_chunk((gg // CH + 1) % 2,
                               base_e + ((t + db) // CH + 1) * CHW)
                q = gg + AHEAD
                step(gg % NB, (gg // CH) % 2, gg % CH,
                     (q // CH) % 2, q % CH, True, True)

        for g in range(TE, NFULL):
            emit(g)
        for q in range(NFULL - NB, NFULL):
            wait_scatter(q % NB)

        e0 = base_e + NFULL * G
        pltpu.sync_copy(src.at[pl.ds(e0, TAIL)], tsidx)
        pltpu.sync_copy(dst.at[pl.ds(e0, TAIL)], tdidx)
        pltpu.sync_copy(table.at[tsidx], trows)
        pltpu.sync_copy(trows, acc.at[tdidx], add=True)
        count_deg_tail(tdidx, TAIL)

        plsc.subcore_barrier()
        pltpu.sync_copy(acc.at[pl.ds(s * SPAN, SPAN)],
                        out.at[c, pl.ds(s * SPAN, SPAN)])

        @pl.when(s == 0)
        def _otail():
            pltpu.sync_copy(acc.at[pl.ds(REM0, REM)],
                            out.at[c, pl.ds(REM0, REM)])

        pltpu.sync_copy(mydeg, deg_out.at[pl.ds(wid * N, N)])

    return k


def _dot(a, b):
    return jnp.dot(a, b, precision=lax.Precision.HIGHEST,
                   preferred_element_type=jnp.float32)


BN = 2000


def _tc_call(body, d_in, d_out):
    return pl.pallas_call(
        body,
        grid=(N // BN,),
        in_specs=[
            pl.BlockSpec((BN, d_in), lambda i: (i, 0)),
            pl.BlockSpec((NC, BN, W), lambda i: (0, i, 0)),
            pl.BlockSpec((BN, NW), lambda i: (i, 0)),
            pl.BlockSpec((d_in, d_out), lambda i: (0, 0)),
            pl.BlockSpec((d_in, d_out), lambda i: (0, 0)),
            pl.BlockSpec((1, d_out), lambda i: (0, 0)),
        ],
        out_specs=pl.BlockSpec((BN, d_out), lambda i: (i, 0)),
        out_shape=jax.ShapeDtypeStruct((N, d_out), jnp.float32),
    )


def _mean(p_ref, dp_ref, d):
    agg = p_ref[0] + p_ref[1]
    deg = jnp.maximum(jnp.sum(dp_ref[...], axis=1, keepdims=True), 1.0)
    return agg[:, :d] / deg


def _tc_mid_body(x_ref, p0_ref, dp0_ref, ws_ref, wn_ref, b_ref, h1_ref):
    mean = _mean(p0_ref, dp0_ref, D_IN)
    h1_ref[...] = jax.nn.relu(
        _dot(x_ref[...], ws_ref[...]) + _dot(mean, wn_ref[...]) + b_ref[...])


def _tc_out_body(h1_ref, p1_ref, dp1_ref, ws_ref, wn_ref, b_ref, o_ref):
    mean = _mean(p1_ref, dp1_ref, D_H)
    o_ref[...] = (_dot(h1_ref[...], ws_ref[...]) + _dot(mean, wn_ref[...])
                  + b_ref[...])


def kernel(x, edge_index0, edge_index1, W_self0, W_neigh0, b0,
           W_self1, W_neigh1, b1):
    src0, dst0 = edge_index0[0], edge_index0[1]
    src1, dst1 = edge_index1[0], edge_index1[1]
    b0r = b0.reshape(1, D_H)
    b1r = b1.reshape(1, D_OUT)
    sc = _sc_segment_sum()

    zs = jnp.zeros((ZCH, W), jnp.float32)
    p0, degf0 = sc(x, src0, dst0, zs)
    dp0 = degf0.reshape(NW, N).T

    h1 = _tc_call(_tc_mid_body, D_IN, D_H)(x, p0, dp0, W_self0, W_neigh0, b0r)

    p1, degf1 = sc(h1, src1, dst1, zs)
    dp1 = degf1.reshape(NW, N).T

    out = _tc_call(_tc_out_body, D_H, D_OUT)(h1, p1, dp1, W_self1, W_neigh1,
                                             b1r)

    return out
